# baseline (speedup 1.0000x reference)
import functools

import jax
import jax.numpy as jnp
from jax import lax
from jax.experimental import pallas as pl
from jax.experimental.pallas import tpu as pltpu
from jax.experimental.pallas import tpu_sc as plsc

_N = 100000
_E = 1600000
_G = 1024
_CH = 4096
_NCH = 25
_NPAD = _CH * _NCH
_EB = 128
_NB = _E // _EB
_BLK = 1024


def _sc_prop(u, esrc, edst, ec):
    mesh = plsc.VectorSubcoreMesh(core_axis_name="c", subcore_axis_name="s")

    @functools.partial(
        pl.kernel,
        mesh=mesh,
        out_type=jax.ShapeDtypeStruct((_NPAD, 128), jnp.float32),
        scratch_types=[
            pltpu.VMEM((48,), jnp.int32),
            pltpu.VMEM((_EB,), jnp.int32),
            pltpu.VMEM((_EB,), jnp.int32),
            pltpu.VMEM((_EB,), jnp.int32),
            pltpu.VMEM((_EB, 128), jnp.float32),
            pltpu.VMEM_SHARED((_CH + 8, 128), jnp.float32),
            pltpu.SemaphoreType.DMA,
        ],
    )
    def kern(u_hbm, esrc_hbm, edst_hbm, ec_hbm, out_hbm,
             ec_v, sidx, draw, didx, rows, acc, sem):
        cid = lax.axis_index("c")
        sid = lax.axis_index("s")
        pltpu.sync_copy(ec_hbm, ec_v)
        nch_mine = (_NCH - cid + 1) // 2

        def chunk_body(ci, carry):
            c = cid + 2 * ci
            base = c * _CH
            pltpu.sync_copy(u_hbm.at[pl.ds(base + sid * 256, 256)],
                            acc.at[pl.ds(sid * 256, 256)])
            plsc.subcore_barrier()
            ev = ec_v[pl.ds(c, 16)]
            e0 = ev[0]
            e1 = ev[1]
            kb0 = e0 // _EB
            kb1 = (e1 + _EB - 1) // _EB
            cnt = jnp.maximum((kb1 - (kb0 + sid) + 15) // 16, 0)

            def batch_body(i, carry2):
                kb = kb0 + sid + 16 * i
                pltpu.sync_copy(esrc_hbm.at[kb], sidx)
                pltpu.sync_copy(edst_hbm.at[kb], draw)
                for j in range(_EB // 16):
                    dv = draw[pl.ds(j * 16, 16)] - base
                    ok = (dv >= 0) & (dv < _CH)
                    didx[pl.ds(j * 16, 16)] = jnp.where(ok, dv, _CH)
                pltpu.async_copy(u_hbm.at[sidx], rows, sem).wait()
                pltpu.sync_copy(rows, acc.at[didx], add=True)
                return carry2

            lax.fori_loop(0, cnt, batch_body, 0)
            plsc.subcore_barrier()
            pltpu.sync_copy(acc.at[pl.ds(sid * 256, 256)],
                            out_hbm.at[pl.ds(base + sid * 256, 256)])
            plsc.subcore_barrier()
            return carry

        lax.fori_loop(0, nch_mine, chunk_body, 0)

    return kern(u, esrc, edst, ec)


def _mm_first(node, W, dis):
    def body(nref, wref, dref, oref):
        oref[...] = jnp.dot(nref[...], wref[...],
                            preferred_element_type=jnp.float32) * dref[...]

    return pl.pallas_call(
        body,
        grid=(_NPAD // _BLK,),
        in_specs=[
            pl.BlockSpec((_BLK, node.shape[1]), lambda i: (i, 0)),
            pl.BlockSpec((node.shape[1], 128), lambda i: (0, 0)),
            pl.BlockSpec((_BLK, 1), lambda i: (i, 0)),
        ],
        out_specs=pl.BlockSpec((_BLK, 128), lambda i: (i, 0)),
        out_shape=jax.ShapeDtypeStruct((_NPAD, 128), jnp.float32),
    )(node, W, dis)


def _mm_mid(acc, W, dis, b):
    def body(aref, wref, dref, bref, oref):
        h = jnp.maximum(aref[...] * dref[...] + bref[...], 0.0)
        oref[...] = jnp.dot(h, wref[...],
                            preferred_element_type=jnp.float32) * dref[...]

    return pl.pallas_call(
        body,
        grid=(_NPAD // _BLK,),
        in_specs=[
            pl.BlockSpec((_BLK, 128), lambda i: (i, 0)),
            pl.BlockSpec((128, 128), lambda i: (0, 0)),
            pl.BlockSpec((_BLK, 1), lambda i: (i, 0)),
            pl.BlockSpec((1, 128), lambda i: (0, 0)),
        ],
        out_specs=pl.BlockSpec((_BLK, 128), lambda i: (i, 0)),
        out_shape=jax.ShapeDtypeStruct((_NPAD, 128), jnp.float32),
    )(acc, W, dis, b)


def kernel(x, edge_index, edge_attr, batch, atom_table, bond_table, bool_table, Wn, bn, We, be, W1, b1, W2, b2):
    xi = x.astype(jnp.int32)
    node = jnp.concatenate([
        atom_table[xi[:, 0]],
        x[:, 1:11] @ Wn.T + bn,
        bool_table[xi[:, -3]],
        bool_table[xi[:, -2]],
        bool_table[xi[:, -1]],
    ], axis=1)
    node = jnp.pad(node, ((0, _NPAD - _N), (0, 0)))

    src = edge_index[0]
    dst = edge_index[1]
    dst_s, src_s = lax.sort([dst, src], num_keys=1)
    row_start = jnp.searchsorted(dst_s, jnp.arange(_N + 1, dtype=jnp.int32))
    deg = (row_start[1:] - row_start[:-1]).astype(jnp.float32) + 1.0
    dis = lax.rsqrt(deg)
    dis_pad = jnp.pad(dis, (0, _NPAD - _N), constant_values=1.0)[:, None]

    ec = jnp.searchsorted(
        dst_s, jnp.arange(0, _NPAD + 1, _CH, dtype=jnp.int32)
    ).astype(jnp.int32)
    ec = jnp.pad(ec, (0, 48 - ec.shape[0]))
    esrc = src_s.reshape(_NB, _EB)
    edst = dst_s.reshape(_NB, _EB)

    u = _mm_first(node, W1, dis_pad)
    a = _sc_prop(u, esrc, edst, ec)
    u = _mm_mid(a, W2, dis_pad, b1[None, :])
    a = _sc_prop(u, esrc, edst, ec)
    u = _mm_mid(a, W2, dis_pad, b2[None, :])
    a = _sc_prop(u, esrc, edst, ec)

    h = jnp.maximum(a[:_N] * dis[:, None] + b2, 0.0)
    return jax.ops.segment_max(h, batch, num_segments=_G)


# speedup vs baseline: 1.0235x; 1.0235x over previous
import functools

import jax
import jax.numpy as jnp
from jax import lax
from jax.experimental import pallas as pl
from jax.experimental.pallas import tpu as pltpu
from jax.experimental.pallas import tpu_sc as plsc

_N = 100000
_E = 1600000
_G = 1024
_CH = 4096
_NCH = 25
_NPAD = _CH * _NCH
_EB = 128
_NB = _E // _EB
_BLK = 1024


def _sc_prop(u, esrc, edst, ec):
    mesh = plsc.VectorSubcoreMesh(core_axis_name="c", subcore_axis_name="s")
    GB = 64
    ND = 4

    @functools.partial(
        pl.kernel,
        mesh=mesh,
        out_type=jax.ShapeDtypeStruct((_NPAD, 128), jnp.float32),
        scratch_types=[
            pltpu.VMEM((48,), jnp.int32),
            pltpu.VMEM((GB, _EB), jnp.int32),
            pltpu.VMEM((GB, _EB), jnp.int32),
            pltpu.VMEM((_EB,), jnp.int32),
            pltpu.VMEM((ND, _EB, 128), jnp.float32),
            pltpu.VMEM_SHARED((_CH + 8, 128), jnp.float32),
            pltpu.SemaphoreType.DMA,
            pltpu.SemaphoreType.DMA,
            pltpu.SemaphoreType.DMA,
            pltpu.SemaphoreType.DMA,
        ],
    )
    def kern(u_hbm, esrc_hbm, edst_hbm, ec_hbm, out_hbm,
             ec_v, sbulk, dbulk, didx, rows, acc, *sems):
        cid = lax.axis_index("c")
        sid = lax.axis_index("s")
        pltpu.sync_copy(ec_hbm, ec_v)
        nch_mine = (_NCH - cid + 1) // 2

        def fire(b, r):
            pltpu.async_copy(u_hbm.at[sbulk.at[b]], rows.at[r], sems[r])

        def drain(b, r):
            pltpu.make_async_copy(u_hbm.at[sbulk.at[b]], rows.at[r],
                                  sems[r]).wait()

        def chunk_body(ci, carry):
            c = cid + 2 * ci
            base = c * _CH
            pltpu.sync_copy(u_hbm.at[pl.ds(base + sid * 256, 256)],
                            acc.at[pl.ds(sid * 256, 256)])
            plsc.subcore_barrier()
            ev = ec_v[pl.ds(c, 16)]
            kb0 = (ev[0] // _EB) // 8 * 8
            kb1 = (ev[1] + _EB - 1) // _EB
            nb = kb1 - kb0
            per = (nb + 127) // 128 * 8
            my0 = kb0 + sid * per
            myn = jnp.maximum(jnp.minimum(per, kb1 - my0), 0)
            ngrp = (myn + GB - 1) // GB

            def group_body(g, carry2):
                g0 = my0 + g * GB
                gn = jnp.minimum(GB, myn - g * GB)
                pltpu.sync_copy(esrc_hbm.at[pl.ds(g0, GB)], sbulk)
                pltpu.sync_copy(edst_hbm.at[pl.ds(g0, GB)], dbulk)
                for r in range(ND - 1):
                    @pl.when(r < gn)
                    def _():
                        fire(r, r)

                def batch_body(i, carry3):
                    for r in range(ND):
                        @pl.when(((i + ND - 1) % ND == r) & (i + ND - 1 < gn))
                        def _():
                            fire(i + ND - 1, r)
                    for r in range(ND):
                        @pl.when(i % ND == r)
                        def _():
                            drain(i, r)
                            for j in range(_EB // 16):
                                dv = dbulk[i, pl.ds(j * 16, 16)] - base
                                ok = (dv >= 0) & (dv < _CH)
                                didx[pl.ds(j * 16, 16)] = jnp.where(ok, dv, _CH)
                            pltpu.sync_copy(rows.at[r], acc.at[didx], add=True)
                    return carry3

                lax.fori_loop(0, gn, batch_body, 0)
                return carry2

            lax.fori_loop(0, ngrp, group_body, 0)
            plsc.subcore_barrier()
            pltpu.sync_copy(acc.at[pl.ds(sid * 256, 256)],
                            out_hbm.at[pl.ds(base + sid * 256, 256)])
            plsc.subcore_barrier()
            return carry

        lax.fori_loop(0, nch_mine, chunk_body, 0)

    return kern(u, esrc, edst, ec)


def _mm_first(node, W, dis):
    def body(nref, wref, dref, oref):
        oref[...] = jnp.dot(nref[...], wref[...],
                            preferred_element_type=jnp.float32) * dref[...]

    return pl.pallas_call(
        body,
        grid=(_NPAD // _BLK,),
        in_specs=[
            pl.BlockSpec((_BLK, node.shape[1]), lambda i: (i, 0)),
            pl.BlockSpec((node.shape[1], 128), lambda i: (0, 0)),
            pl.BlockSpec((_BLK, 1), lambda i: (i, 0)),
        ],
        out_specs=pl.BlockSpec((_BLK, 128), lambda i: (i, 0)),
        out_shape=jax.ShapeDtypeStruct((_NPAD, 128), jnp.float32),
    )(node, W, dis)


def _mm_mid(acc, W, dis, b):
    def body(aref, wref, dref, bref, oref):
        h = jnp.maximum(aref[...] * dref[...] + bref[...], 0.0)
        oref[...] = jnp.dot(h, wref[...],
                            preferred_element_type=jnp.float32) * dref[...]

    return pl.pallas_call(
        body,
        grid=(_NPAD // _BLK,),
        in_specs=[
            pl.BlockSpec((_BLK, 128), lambda i: (i, 0)),
            pl.BlockSpec((128, 128), lambda i: (0, 0)),
            pl.BlockSpec((_BLK, 1), lambda i: (i, 0)),
            pl.BlockSpec((1, 128), lambda i: (0, 0)),
        ],
        out_specs=pl.BlockSpec((_BLK, 128), lambda i: (i, 0)),
        out_shape=jax.ShapeDtypeStruct((_NPAD, 128), jnp.float32),
    )(acc, W, dis, b)


def kernel(x, edge_index, edge_attr, batch, atom_table, bond_table, bool_table, Wn, bn, We, be, W1, b1, W2, b2):
    xi = x.astype(jnp.int32)
    node = jnp.concatenate([
        atom_table[xi[:, 0]],
        x[:, 1:11] @ Wn.T + bn,
        bool_table[xi[:, -3]],
        bool_table[xi[:, -2]],
        bool_table[xi[:, -1]],
    ], axis=1)
    node = jnp.pad(node, ((0, _NPAD - _N), (0, 0)))

    src = edge_index[0]
    dst = edge_index[1]
    dst_s, src_s = lax.sort([dst, src], num_keys=1)
    row_start = jnp.searchsorted(dst_s, jnp.arange(_N + 1, dtype=jnp.int32))
    deg = (row_start[1:] - row_start[:-1]).astype(jnp.float32) + 1.0
    dis = lax.rsqrt(deg)
    dis_pad = jnp.pad(dis, (0, _NPAD - _N), constant_values=1.0)[:, None]

    ec = jnp.searchsorted(
        dst_s, jnp.arange(0, _NPAD + 1, _CH, dtype=jnp.int32)
    ).astype(jnp.int32)
    ec = jnp.pad(ec, (0, 48 - ec.shape[0]))
    esrc = jnp.pad(src_s.reshape(_NB, _EB), ((0, 64), (0, 0)))
    edst = jnp.pad(dst_s.reshape(_NB, _EB), ((0, 64), (0, 0)))

    u = _mm_first(node, W1, dis_pad)
    a = _sc_prop(u, esrc, edst, ec)
    u = _mm_mid(a, W2, dis_pad, b1[None, :])
    a = _sc_prop(u, esrc, edst, ec)
    u = _mm_mid(a, W2, dis_pad, b2[None, :])
    a = _sc_prop(u, esrc, edst, ec)

    h = jnp.maximum(a[:_N] * dis[:, None] + b2, 0.0)
    return jax.ops.segment_max(h, batch, num_segments=_G)


# speedup vs baseline: 1.0255x; 1.0020x over previous
import functools

import jax
import jax.numpy as jnp
from jax import lax
from jax.experimental import pallas as pl
from jax.experimental.pallas import tpu as pltpu
from jax.experimental.pallas import tpu_sc as plsc

_N = 100000
_E = 1600000
_G = 1024
_CH = 4096
_NCH = 25
_NPAD = _CH * _NCH
_EB = 128
_NB = _E // _EB
_BLK = 1024


def _sc_prop(u, esrc, edst, ec):
    mesh = plsc.VectorSubcoreMesh(core_axis_name="c", subcore_axis_name="s")
    GB = 64
    ND = 4

    @functools.partial(
        pl.kernel,
        mesh=mesh,
        out_type=jax.ShapeDtypeStruct((_NPAD, 128), jnp.float32),
        scratch_types=[
            pltpu.VMEM((48,), jnp.int32),
            pltpu.VMEM((GB, _EB), jnp.int32),
            pltpu.VMEM((GB, _EB), jnp.int32),
            pltpu.VMEM((ND, _EB), jnp.int32),
            pltpu.VMEM((ND, _EB, 128), jnp.float32),
            pltpu.VMEM_SHARED((_CH + 8, 128), jnp.float32),
            pltpu.SemaphoreType.DMA,
            pltpu.SemaphoreType.DMA,
            pltpu.SemaphoreType.DMA,
            pltpu.SemaphoreType.DMA,
            pltpu.SemaphoreType.DMA,
            pltpu.SemaphoreType.DMA,
            pltpu.SemaphoreType.DMA,
            pltpu.SemaphoreType.DMA,
        ],
    )
    def kern(u_hbm, esrc_hbm, edst_hbm, ec_hbm, out_hbm,
             ec_v, sbulk, dbulk, didx, rows, acc, *sems):
        cid = lax.axis_index("c")
        sid = lax.axis_index("s")
        pltpu.sync_copy(ec_hbm, ec_v)
        nch_mine = (_NCH - cid + 1) // 2

        gsems = sems[:ND]
        ssems = sems[ND:]

        def fire(b, r):
            pltpu.async_copy(u_hbm.at[sbulk.at[b]], rows.at[r], gsems[r])

        def drain(b, r):
            pltpu.make_async_copy(u_hbm.at[sbulk.at[b]], rows.at[r],
                                  gsems[r]).wait()

        def sdrain(r):
            pltpu.make_async_copy(rows.at[r], acc.at[didx.at[r]],
                                  ssems[r]).wait()

        def chunk_body(ci, carry):
            c = cid + 2 * ci
            base = c * _CH
            pltpu.sync_copy(u_hbm.at[pl.ds(base + sid * 256, 256)],
                            acc.at[pl.ds(sid * 256, 256)])
            plsc.subcore_barrier()
            ev = ec_v[pl.ds(c, 16)]
            kb0 = (ev[0] // _EB) // 8 * 8
            kb1 = (ev[1] + _EB - 1) // _EB
            nb = kb1 - kb0
            per = (nb + 127) // 128 * 8
            my0 = kb0 + sid * per
            myn = jnp.maximum(jnp.minimum(per, kb1 - my0), 0)
            ngrp = (myn + GB - 1) // GB

            def group_body(g, carry2):
                g0 = my0 + g * GB
                gn = jnp.minimum(GB, myn - g * GB)
                pltpu.sync_copy(esrc_hbm.at[pl.ds(g0, GB)], sbulk)
                pltpu.sync_copy(edst_hbm.at[pl.ds(g0, GB)], dbulk)
                for r in range(ND - 1):
                    @pl.when(r < gn)
                    def _():
                        fire(r, r)

                def batch_body(i, carry3):
                    for r in range(ND):
                        @pl.when(((i + ND - 1) % ND == r) & (i + ND - 1 < gn))
                        def _():
                            @pl.when(i >= 1)
                            def _():
                                sdrain(r)
                            fire(i + ND - 1, r)
                    for r in range(ND):
                        @pl.when(i % ND == r)
                        def _():
                            drain(i, r)
                            for j in range(_EB // 16):
                                dv = dbulk[i, pl.ds(j * 16, 16)] - base
                                ok = (dv >= 0) & (dv < _CH)
                                didx[r, pl.ds(j * 16, 16)] = jnp.where(
                                    ok, dv, _CH)
                            pltpu.async_copy(rows.at[r], acc.at[didx.at[r]],
                                             ssems[r], add=True)
                    return carry3

                lax.fori_loop(0, gn, batch_body, 0)
                for r in range(ND):
                    @pl.when(r < gn)
                    def _():
                        sdrain(r)
                return carry2

            lax.fori_loop(0, ngrp, group_body, 0)
            plsc.subcore_barrier()
            pltpu.sync_copy(acc.at[pl.ds(sid * 256, 256)],
                            out_hbm.at[pl.ds(base + sid * 256, 256)])
            plsc.subcore_barrier()
            return carry

        lax.fori_loop(0, nch_mine, chunk_body, 0)

    return kern(u, esrc, edst, ec)


def _mm_first(node, W, dis):
    def body(nref, wref, dref, oref):
        oref[...] = jnp.dot(nref[...], wref[...],
                            preferred_element_type=jnp.float32) * dref[...]

    return pl.pallas_call(
        body,
        grid=(_NPAD // _BLK,),
        in_specs=[
            pl.BlockSpec((_BLK, node.shape[1]), lambda i: (i, 0)),
            pl.BlockSpec((node.shape[1], 128), lambda i: (0, 0)),
            pl.BlockSpec((_BLK, 1), lambda i: (i, 0)),
        ],
        out_specs=pl.BlockSpec((_BLK, 128), lambda i: (i, 0)),
        out_shape=jax.ShapeDtypeStruct((_NPAD, 128), jnp.float32),
    )(node, W, dis)


def _mm_mid(acc, W, dis, b):
    def body(aref, wref, dref, bref, oref):
        h = jnp.maximum(aref[...] * dref[...] + bref[...], 0.0)
        oref[...] = jnp.dot(h, wref[...],
                            preferred_element_type=jnp.float32) * dref[...]

    return pl.pallas_call(
        body,
        grid=(_NPAD // _BLK,),
        in_specs=[
            pl.BlockSpec((_BLK, 128), lambda i: (i, 0)),
            pl.BlockSpec((128, 128), lambda i: (0, 0)),
            pl.BlockSpec((_BLK, 1), lambda i: (i, 0)),
            pl.BlockSpec((1, 128), lambda i: (0, 0)),
        ],
        out_specs=pl.BlockSpec((_BLK, 128), lambda i: (i, 0)),
        out_shape=jax.ShapeDtypeStruct((_NPAD, 128), jnp.float32),
    )(acc, W, dis, b)


def kernel(x, edge_index, edge_attr, batch, atom_table, bond_table, bool_table, Wn, bn, We, be, W1, b1, W2, b2):
    xi = x.astype(jnp.int32)
    node = jnp.concatenate([
        atom_table[xi[:, 0]],
        x[:, 1:11] @ Wn.T + bn,
        bool_table[xi[:, -3]],
        bool_table[xi[:, -2]],
        bool_table[xi[:, -1]],
    ], axis=1)
    node = jnp.pad(node, ((0, _NPAD - _N), (0, 0)))

    src = edge_index[0]
    dst = edge_index[1]
    dst_s, src_s = lax.sort([dst, src], num_keys=1)
    row_start = jnp.searchsorted(dst_s, jnp.arange(_N + 1, dtype=jnp.int32))
    deg = (row_start[1:] - row_start[:-1]).astype(jnp.float32) + 1.0
    dis = lax.rsqrt(deg)
    dis_pad = jnp.pad(dis, (0, _NPAD - _N), constant_values=1.0)[:, None]

    ec = jnp.searchsorted(
        dst_s, jnp.arange(0, _NPAD + 1, _CH, dtype=jnp.int32)
    ).astype(jnp.int32)
    ec = jnp.pad(ec, (0, 48 - ec.shape[0]))
    esrc = jnp.pad(src_s.reshape(_NB, _EB), ((0, 64), (0, 0)))
    edst = jnp.pad(dst_s.reshape(_NB, _EB), ((0, 64), (0, 0)))

    u = _mm_first(node, W1, dis_pad)
    a = _sc_prop(u, esrc, edst, ec)
    u = _mm_mid(a, W2, dis_pad, b1[None, :])
    a = _sc_prop(u, esrc, edst, ec)
    u = _mm_mid(a, W2, dis_pad, b2[None, :])
    a = _sc_prop(u, esrc, edst, ec)

    h = jnp.maximum(a[:_N] * dis[:, None] + b2, 0.0)
    return jax.ops.segment_max(h, batch, num_segments=_G)
